# baseline (speedup 1.0000x reference)
import functools

import jax
import jax.numpy as jnp
from jax import lax
from jax.experimental import pallas as pl
from jax.experimental.pallas import tpu as pltpu
from jax.experimental.pallas import tpu_sc as plsc

NC = 2
NS = 16
NW = NC * NS
LANES = 128
IB = 8
IBD = 8
NBUF = 4
DEG_D = 16
TRASH = 64


def _bin_sc(N, R, NH, TR0):
    mesh = plsc.VectorSubcoreMesh(
        core_axis_name="c", subcore_axis_name="s",
        num_cores=NC, num_subcores=NS)
    RB = R // NW
    ED = RB * LANES

    @functools.partial(
        pl.kernel,
        out_type=(
            jax.ShapeDtypeStruct((NC * R * LANES,), jnp.int32),
            jax.ShapeDtypeStruct((NC * R * LANES,), jnp.int32),
            jax.ShapeDtypeStruct((NW * 8, LANES), jnp.int32),
        ),
        mesh=mesh,
        compiler_params=pltpu.CompilerParams(needs_layout_passes=False),
        scratch_types=[
            pltpu.VMEM((RB, LANES), jnp.int32),
            pltpu.VMEM((RB, LANES), jnp.int32),
            pltpu.VMEM((ED + 16,), jnp.int32),
            pltpu.VMEM((ED + 16,), jnp.int32),
            pltpu.VMEM((ED + 16,), jnp.int32),
            pltpu.VMEM((ED + 16,), jnp.int32),
            pltpu.VMEM((8, LANES), jnp.int32),
        ],
    )
    def bink(src_hbm, dst_hbm, srcb, dstb, cnts, idxs, idxd,
             s0, d0, s1, d1, cntb):
        c = lax.axis_index("c")
        s = lax.axis_index("s")
        wid = c * NS + s
        zero16 = jnp.zeros((16,), jnp.int32)
        trash16 = jnp.full((16,), TR0, jnp.int32)
        nh16 = jnp.full((16,), NH, jnp.int32)
        n16 = jnp.full((16,), N, jnp.int32)

        def pf(i, carry):
            s0[pl.ds(i * 16, 16)] = zero16
            d0[pl.ds(i * 16, 16)] = trash16
            s1[pl.ds(i * 16, 16)] = zero16
            d1[pl.ds(i * 16, 16)] = trash16
            return carry
        lax.fori_loop(0, (ED + 16) // 16, pf, 0)

        def zc(i, carry):
            for k in range(LANES // 16):
                cntb[i, pl.ds(k * 16, 16)] = zero16
            return carry
        lax.fori_loop(0, 8, zc, 0)

        pltpu.sync_copy(src_hbm.at[pl.ds(wid * RB, RB)], idxs)
        pltpu.sync_copy(dst_hbm.at[pl.ds(wid * RB, RB)], idxd)

        def row(r, offs):
            off0, off1 = offs
            for k in range(LANES // 16):
                vs = idxs[r, pl.ds(k * 16, 16)]
                vd = idxd[r, pl.ds(k * 16, 16)]
                m0 = vd < nh16
                m1 = (vd >= nh16) & (vd < n16)
                plsc.store_compressed(s0.at[pl.ds(off0, 16)], vs, mask=m0)
                plsc.store_compressed(d0.at[pl.ds(off0, 16)], vd, mask=m0)
                plsc.store_compressed(s1.at[pl.ds(off1, 16)], vs, mask=m1)
                plsc.store_compressed(d1.at[pl.ds(off1, 16)], vd - nh16, mask=m1)
                off0 = off0 + jnp.max(plsc.all_reduce_population_count(m0))
                off1 = off1 + jnp.max(plsc.all_reduce_population_count(m1))
            return off0, off1
        off0, off1 = lax.fori_loop(
            0, RB, row, (jnp.int32(0), jnp.int32(0)))

        one16 = jnp.full((16,), 1, jnp.int32)
        cntb[0, pl.ds(0, 16)] = one16 * off0
        cntb[0, pl.ds(16, 16)] = one16 * off1

        pltpu.sync_copy(s0.at[pl.ds(0, ED)], srcb.at[pl.ds(wid * ED, ED)])
        pltpu.sync_copy(d0.at[pl.ds(0, ED)], dstb.at[pl.ds(wid * ED, ED)])
        pltpu.sync_copy(s1.at[pl.ds(0, ED)],
                        srcb.at[pl.ds(R * LANES + wid * ED, ED)])
        pltpu.sync_copy(d1.at[pl.ds(0, ED)],
                        dstb.at[pl.ds(R * LANES + wid * ED, ED)])
        pltpu.sync_copy(cntb, cnts.at[pl.ds(wid * 8, 8)])

    return bink


def _spmm_sc(N, D, R, NH, TR0, acc_rows, z_rows):
    mesh = plsc.VectorSubcoreMesh(
        core_axis_name="c", subcore_axis_name="s",
        num_cores=NC, num_subcores=NS)
    RB = R // NW

    @functools.partial(
        pl.kernel,
        out_type=jax.ShapeDtypeStruct((NC, acc_rows, D), jnp.float32),
        mesh=mesh,
        compiler_params=pltpu.CompilerParams(needs_layout_passes=False),
        scratch_types=[
            pltpu.VMEM((IB, LANES), jnp.int32),
            pltpu.VMEM((IB, LANES), jnp.int32),
            pltpu.VMEM((NBUF, LANES, D), jnp.float32),
            pltpu.VMEM((z_rows, D), jnp.float32),
            pltpu.VMEM((16, LANES), jnp.int32),
            pltpu.VMEM_SHARED((acc_rows, D), jnp.float32),
            pltpu.SemaphoreType.DMA,
            pltpu.SemaphoreType.DMA,
            pltpu.SemaphoreType.DMA,
        ],
    )
    def spmm(g_hbm, srcb_hbm, dstb_hbm, cnt_hbm, out_hbm, idxs, idxd, rows,
             zbuf, cntv, acc, gsem, ssem0, ssem1):
        c = lax.axis_index("c")
        s = lax.axis_index("s")
        rows_per_tile = acc_rows // NS
        ssems = (ssem0, ssem1)

        def zrow(r, carry):
            for j in range(D // 16):
                zbuf[r, pl.ds(j * 16, 16)] = jnp.zeros((16,), jnp.float32)
            return carry
        lax.fori_loop(0, z_rows, zrow, 0)

        def zcopy(i, carry):
            pltpu.sync_copy(
                zbuf, acc.at[pl.ds(s * rows_per_tile + i * z_rows, z_rows)])
            return carry
        lax.fori_loop(0, rows_per_tile // z_rows, zcopy, 0)
        pltpu.sync_copy(cnt_hbm.at[pl.ds(s * 16, 16)], cntv)
        plsc.subcore_barrier()

        def step_for(region_base):
            def step(i, carry):
                rb = region_base + i * IB
                pltpu.sync_copy(srcb_hbm.at[c, pl.ds(rb, IB)], idxs)
                pltpu.sync_copy(dstb_hbm.at[c, pl.ds(rb, IB)], idxd)
                pending = {}
                for grp in range(IB // 2):
                    pp = grp & 1
                    b0, b1 = 2 * pp, 2 * pp + 1
                    r0, r1 = 2 * grp, 2 * grp + 1
                    for d in pending.pop(pp, ()):
                        d.wait()
                    gd0 = pltpu.async_copy(g_hbm.at[idxs.at[r0]],
                                           rows.at[b0], gsem)
                    gd1 = pltpu.async_copy(g_hbm.at[idxs.at[r1]],
                                           rows.at[b1], gsem)
                    gd0.wait()
                    gd1.wait()
                    pending[pp] = (
                        pltpu.async_copy(rows.at[b0], acc.at[idxd.at[r0]],
                                         ssems[pp], add=True),
                        pltpu.async_copy(rows.at[b1], acc.at[idxd.at[r1]],
                                         ssems[pp], add=True),
                    )
                for pp in sorted(pending):
                    for d in pending[pp]:
                        d.wait()
                return carry
            return step

        for reg in range(2):
            w = 2 * s + reg
            lax.fori_loop(0, RB // IB, step_for(w * RB), 0)
        plsc.subcore_barrier()

        pltpu.sync_copy(acc.at[pl.ds(s * rows_per_tile, rows_per_tile)],
                        out_hbm.at[c, pl.ds(s * rows_per_tile, rows_per_tile)])

    return spmm


def _deg_sc(N, R, hp_rows):
    mesh = plsc.VectorSubcoreMesh(
        core_axis_name="c", subcore_axis_name="s",
        num_cores=NC, num_subcores=NS)
    RW = R // NW

    @functools.partial(
        pl.kernel,
        out_type=jax.ShapeDtypeStruct((NW, hp_rows, LANES), jnp.float32),
        mesh=mesh,
        compiler_params=pltpu.CompilerParams(needs_layout_passes=False),
        scratch_types=[
            pltpu.VMEM((IBD, LANES), jnp.int32),
            pltpu.VMEM((hp_rows, LANES), jnp.float32),
        ],
    )
    def deg(dst_hbm, out_hbm, idxd, histo):
        c = lax.axis_index("c")
        s = lax.axis_index("s")
        wid = c * NS + s

        def zrow(r, carry):
            for k in range(LANES // 16):
                histo[r, pl.ds(k * 16, 16)] = jnp.zeros((16,), jnp.float32)
            return carry
        lax.fori_loop(0, hp_rows, zrow, 0)

        ones16 = jnp.ones((16,), jnp.float32)

        def step(i, carry):
            rb = wid * RW + i * IBD
            pltpu.sync_copy(dst_hbm.at[pl.ds(rb, IBD)], idxd)
            for r in range(IBD):
                for k in range(LANES // 16):
                    v = idxd[r, pl.ds(k * 16, 16)]
                    rowi = lax.shift_right_logical(v, 7)
                    coli = v & (LANES - 1)
                    plsc.addupdate_scatter(histo, [rowi, coli], ones16)
            return carry
        lax.fori_loop(0, RW // IBD, step, 0)

        pltpu.sync_copy(histo, out_hbm.at[wid])

    return deg


def _tc_first(N, Din, Dh, BR):
    def body(x_ref, w_ref, dp_ref, o_ref):
        deg = jnp.sum(dp_ref[...], axis=1, keepdims=True) + 1.0
        dis = lax.rsqrt(deg)
        h = jnp.dot(x_ref[...], w_ref[...], preferred_element_type=jnp.float32)
        o_ref[...] = h * dis

    return pl.pallas_call(
        body,
        grid=(N // BR,),
        in_specs=[
            pl.BlockSpec((BR, Din), lambda i: (i, 0)),
            pl.BlockSpec((Din, Dh), lambda i: (0, 0)),
            pl.BlockSpec((BR, NW), lambda i: (i, 0)),
        ],
        out_specs=pl.BlockSpec((BR, Dh), lambda i: (i, 0)),
        out_shape=jax.ShapeDtypeStruct((N, Dh), jnp.float32),
    )


def _tc_mid(N, Dh, Dn, BR, BPC):
    def body(p_ref, g_ref, dp_ref, w_ref, b_ref, o_ref):
        deg = jnp.sum(dp_ref[...], axis=1, keepdims=True) + 1.0
        dis = lax.rsqrt(deg)
        t = (p_ref[0] + g_ref[...]) * dis + b_ref[...]
        a = jnp.maximum(t, 0.0)
        h = jnp.dot(a, w_ref[...], preferred_element_type=jnp.float32)
        g2 = h * dis
        o_ref[...] = jnp.concatenate(
            [g2, jnp.zeros((g2.shape[0], Dh - Dn), jnp.float32)], axis=-1)

    return pl.pallas_call(
        body,
        grid=(N // BR,),
        in_specs=[
            pl.BlockSpec((1, BR, Dh), lambda i: (i // BPC, i % BPC, 0)),
            pl.BlockSpec((BR, Dh), lambda i: (i, 0)),
            pl.BlockSpec((BR, NW), lambda i: (i, 0)),
            pl.BlockSpec((Dh, Dn), lambda i: (0, 0)),
            pl.BlockSpec((1, Dh), lambda i: (0, 0)),
        ],
        out_specs=pl.BlockSpec((BR, Dh), lambda i: (i, 0)),
        out_shape=jax.ShapeDtypeStruct((N, Dh), jnp.float32),
    )


def _tc_last(N, Dp, Dh, Do, BR, BPC):
    def body(p_ref, g_ref, dp_ref, b_ref, wo_ref, bo_ref, o_ref):
        deg = jnp.sum(dp_ref[...], axis=1, keepdims=True) + 1.0
        dis = lax.rsqrt(deg)
        t = (p_ref[0][:, :Dh] + g_ref[...][:, :Dh]) * dis + b_ref[...]
        a = jnp.maximum(t, 0.0)
        o_ref[...] = (jnp.dot(a, wo_ref[...], preferred_element_type=jnp.float32)
                      + bo_ref[...])

    return pl.pallas_call(
        body,
        grid=(N // BR,),
        in_specs=[
            pl.BlockSpec((1, BR, Dp), lambda i: (i // BPC, i % BPC, 0)),
            pl.BlockSpec((BR, Dp), lambda i: (i, 0)),
            pl.BlockSpec((BR, NW), lambda i: (i, 0)),
            pl.BlockSpec((1, Dh), lambda i: (0, 0)),
            pl.BlockSpec((Dh, Do), lambda i: (0, 0)),
            pl.BlockSpec((1, Do), lambda i: (0, 0)),
        ],
        out_specs=pl.BlockSpec((BR, Do), lambda i: (i, 0)),
        out_shape=jax.ShapeDtypeStruct((N, Do), jnp.float32),
    )


def kernel(x, edge_index, W1, b1, W2, b2, Wo, bo):
    N, Din = x.shape
    E = edge_index.shape[1]
    Dh1 = W1.shape[1]
    Dh2 = W2.shape[1]
    Do = Wo.shape[1]

    r_raw = -(-E // LANES)
    gran = max(NS * IB, NW * IBD)
    R = -(-r_raw // gran) * gran
    e_pad = R * LANES - E
    src = edge_index[0].astype(jnp.int32)
    dst = edge_index[1].astype(jnp.int32)
    src_p = jnp.concatenate([src, jnp.zeros((e_pad,), jnp.int32)]).reshape(R, LANES)
    dst_p = jnp.concatenate([dst, jnp.full((e_pad,), N, jnp.int32)]).reshape(R, LANES)

    NH = -(-N // NC)
    TR0 = -(-NH // TRASH) * TRASH
    z_rows = 16
    acc_rows = -(-(TR0 + TRASH) // (NS * z_rows)) * NS * z_rows
    hp_rows = -(-(N + 1) // LANES)
    hp_rows += (-hp_rows) % 8

    BR = 1000 if (N // NC) % 1000 == 0 else 500
    BPC = (N // NC) // BR
    b1r = b1.reshape(1, Dh1)
    b2r = b2.reshape(1, Dh2)
    do_pad = max(8, Do)
    wo_p = jnp.zeros((Dh2, do_pad), jnp.float32).at[:, :Do].set(Wo)
    bo_p = jnp.zeros((1, do_pad), jnp.float32).at[0, :Do].set(bo)

    srcb_f, dstb_f, counts = _bin_sc(N, R, NH, TR0)(src_p, dst_p)
    srcb = srcb_f.reshape(NC, R, LANES)
    dstb = dstb_f.reshape(NC, R, LANES)
    deg_parts = _deg_sc(N, R, hp_rows)(dst_p)
    deg_t = jnp.transpose(deg_parts.reshape(NW, hp_rows * LANES))
    g1 = _tc_first(N, Din, Dh1, BR)(x, W1, deg_t)
    p1 = _spmm_sc(N, Dh1, R, NH, TR0, acc_rows, z_rows)(g1, srcb, dstb, counts)
    g2 = _tc_mid(N, Dh1, Dh2, BR, BPC)(p1, g1, deg_t, W2, b1r)
    p2 = _spmm_sc(N, Dh1, R, NH, TR0, acc_rows, z_rows)(g2, srcb, dstb, counts)
    out = _tc_last(N, Dh1, Dh2, do_pad, BR, BPC)(p2, g2, deg_t, b2r, wo_p, bo_p)
    return out[:, :Do]


# speedup vs baseline: 65.3364x; 65.3364x over previous
import functools

import jax
import jax.numpy as jnp
from jax import lax
from jax.experimental import pallas as pl
from jax.experimental.pallas import tpu as pltpu
from jax.experimental.pallas import tpu_sc as plsc

NC = 2
NS = 16
NW = NC * NS
LANES = 128
IB = 8
IBD = 8
NBUF = 4
DEG_D = 16
TRASH = 64


def _bin_sc(N, R, NH, TR0):
    mesh = plsc.VectorSubcoreMesh(
        core_axis_name="c", subcore_axis_name="s",
        num_cores=NC, num_subcores=NS)
    RB = R // NW
    ED = RB * LANES

    @functools.partial(
        pl.kernel,
        out_type=(
            jax.ShapeDtypeStruct((NC * R * LANES,), jnp.int32),
            jax.ShapeDtypeStruct((NC * R * LANES,), jnp.int32),
            jax.ShapeDtypeStruct((NW * 8, LANES), jnp.int32),
        ),
        mesh=mesh,
        compiler_params=pltpu.CompilerParams(needs_layout_passes=False),
        scratch_types=[
            pltpu.VMEM((RB, LANES), jnp.int32),
            pltpu.VMEM((RB, LANES), jnp.int32),
            pltpu.VMEM((ED + 16,), jnp.int32),
            pltpu.VMEM((ED + 16,), jnp.int32),
            pltpu.VMEM((ED + 16,), jnp.int32),
            pltpu.VMEM((ED + 16,), jnp.int32),
            pltpu.VMEM((8, LANES), jnp.int32),
        ],
    )
    def bink(src_hbm, dst_hbm, srcb, dstb, cnts, idxs, idxd,
             s0, d0, s1, d1, cntb):
        c = lax.axis_index("c")
        s = lax.axis_index("s")
        wid = c * NS + s
        zero16 = jnp.zeros((16,), jnp.int32)
        nh16 = jnp.full((16,), NH, jnp.int32)
        n16 = jnp.full((16,), N, jnp.int32)
        iota16 = lax.iota(jnp.int32, 16)

        def pf(i, carry):
            spread_src = (i & 255) * 16 + iota16
            spread_dst = (i & 3) * 16 + iota16 + TR0
            s0[pl.ds(i * 16, 16)] = spread_src
            d0[pl.ds(i * 16, 16)] = spread_dst
            s1[pl.ds(i * 16, 16)] = spread_src
            d1[pl.ds(i * 16, 16)] = spread_dst
            return carry
        lax.fori_loop(0, (ED + 16) // 16, pf, 0)

        def zc(i, carry):
            for k in range(LANES // 16):
                cntb[i, pl.ds(k * 16, 16)] = zero16
            return carry
        lax.fori_loop(0, 8, zc, 0)

        pltpu.sync_copy(src_hbm.at[pl.ds(wid * RB, RB)], idxs)
        pltpu.sync_copy(dst_hbm.at[pl.ds(wid * RB, RB)], idxd)

        def row(r, offs):
            off0, off1 = offs
            for k in range(LANES // 16):
                vs = idxs[r, pl.ds(k * 16, 16)]
                vd = idxd[r, pl.ds(k * 16, 16)]
                m0 = vd < nh16
                m1 = (vd >= nh16) & (vd < n16)
                plsc.store_compressed(s0.at[pl.ds(off0, 16)], vs, mask=m0)
                plsc.store_compressed(d0.at[pl.ds(off0, 16)], vd, mask=m0)
                plsc.store_compressed(s1.at[pl.ds(off1, 16)], vs, mask=m1)
                plsc.store_compressed(d1.at[pl.ds(off1, 16)], vd - nh16, mask=m1)
                off0 = off0 + jnp.max(plsc.all_reduce_population_count(m0))
                off1 = off1 + jnp.max(plsc.all_reduce_population_count(m1))
            return off0, off1
        off0, off1 = lax.fori_loop(
            0, RB, row, (jnp.int32(0), jnp.int32(0)))

        one16 = jnp.full((16,), 1, jnp.int32)
        cntb[0, pl.ds(0, 16)] = one16 * off0
        cntb[0, pl.ds(16, 16)] = one16 * off1

        pltpu.sync_copy(s0.at[pl.ds(0, ED)], srcb.at[pl.ds(wid * ED, ED)])
        pltpu.sync_copy(d0.at[pl.ds(0, ED)], dstb.at[pl.ds(wid * ED, ED)])
        pltpu.sync_copy(s1.at[pl.ds(0, ED)],
                        srcb.at[pl.ds(R * LANES + wid * ED, ED)])
        pltpu.sync_copy(d1.at[pl.ds(0, ED)],
                        dstb.at[pl.ds(R * LANES + wid * ED, ED)])
        pltpu.sync_copy(cntb, cnts.at[pl.ds(wid * 8, 8)])

    return bink


def _spmm_sc(N, D, R, NH, TR0, acc_rows, z_rows):
    mesh = plsc.VectorSubcoreMesh(
        core_axis_name="c", subcore_axis_name="s",
        num_cores=NC, num_subcores=NS)
    RB = R // NW

    @functools.partial(
        pl.kernel,
        out_type=jax.ShapeDtypeStruct((NC, acc_rows, D), jnp.float32),
        mesh=mesh,
        compiler_params=pltpu.CompilerParams(needs_layout_passes=False),
        scratch_types=[
            pltpu.VMEM((IB, LANES), jnp.int32),
            pltpu.VMEM((IB, LANES), jnp.int32),
            pltpu.VMEM((NBUF, LANES, D), jnp.float32),
            pltpu.VMEM((z_rows, D), jnp.float32),
            pltpu.VMEM((16, LANES), jnp.int32),
            pltpu.VMEM_SHARED((acc_rows, D), jnp.float32),
            pltpu.SemaphoreType.DMA,
            pltpu.SemaphoreType.DMA,
            pltpu.SemaphoreType.DMA,
        ],
    )
    def spmm(g_hbm, srcb_hbm, dstb_hbm, cnt_hbm, out_hbm, idxs, idxd, rows,
             zbuf, cntv, acc, gsem, ssem0, ssem1):
        c = lax.axis_index("c")
        s = lax.axis_index("s")
        rows_per_tile = acc_rows // NS
        ssems = (ssem0, ssem1)

        def zrow(r, carry):
            for j in range(D // 16):
                zbuf[r, pl.ds(j * 16, 16)] = jnp.zeros((16,), jnp.float32)
            return carry
        lax.fori_loop(0, z_rows, zrow, 0)

        def zcopy(i, carry):
            pltpu.sync_copy(
                zbuf, acc.at[pl.ds(s * rows_per_tile + i * z_rows, z_rows)])
            return carry
        lax.fori_loop(0, rows_per_tile // z_rows, zcopy, 0)
        pltpu.sync_copy(cnt_hbm.at[pl.ds(s * 16, 16)], cntv)
        plsc.subcore_barrier()

        def step_for(region_base):
            def step(i, carry):
                rb = region_base + i * IB
                pltpu.sync_copy(srcb_hbm.at[c, pl.ds(rb, IB)], idxs)
                pltpu.sync_copy(dstb_hbm.at[c, pl.ds(rb, IB)], idxd)
                pending = {}
                for grp in range(IB // 2):
                    pp = grp & 1
                    b0, b1 = 2 * pp, 2 * pp + 1
                    r0, r1 = 2 * grp, 2 * grp + 1
                    for d in pending.pop(pp, ()):
                        d.wait()
                    gd0 = pltpu.async_copy(g_hbm.at[idxs.at[r0]],
                                           rows.at[b0], gsem)
                    gd1 = pltpu.async_copy(g_hbm.at[idxs.at[r1]],
                                           rows.at[b1], gsem)
                    gd0.wait()
                    gd1.wait()
                    pending[pp] = (
                        pltpu.async_copy(rows.at[b0], acc.at[idxd.at[r0]],
                                         ssems[pp], add=True),
                        pltpu.async_copy(rows.at[b1], acc.at[idxd.at[r1]],
                                         ssems[pp], add=True),
                    )
                for pp in sorted(pending):
                    for d in pending[pp]:
                        d.wait()
                return carry
            return step

        for reg in range(2):
            w = 2 * s + reg
            cnt = jnp.max(cntv[reg * 8, pl.ds(c * 16, 16)])
            nsteps = (cnt + IB * LANES - 1) // (IB * LANES)
            lax.fori_loop(0, nsteps, step_for(w * RB), 0)
        plsc.subcore_barrier()

        pltpu.sync_copy(acc.at[pl.ds(s * rows_per_tile, rows_per_tile)],
                        out_hbm.at[c, pl.ds(s * rows_per_tile, rows_per_tile)])

    return spmm


def _deg_sc(N, R, hp_rows):
    mesh = plsc.VectorSubcoreMesh(
        core_axis_name="c", subcore_axis_name="s",
        num_cores=NC, num_subcores=NS)
    RW = R // NW

    @functools.partial(
        pl.kernel,
        out_type=jax.ShapeDtypeStruct((NW, hp_rows, LANES), jnp.float32),
        mesh=mesh,
        compiler_params=pltpu.CompilerParams(needs_layout_passes=False),
        scratch_types=[
            pltpu.VMEM((IBD, LANES), jnp.int32),
            pltpu.VMEM((hp_rows, LANES), jnp.float32),
        ],
    )
    def deg(dst_hbm, out_hbm, idxd, histo):
        c = lax.axis_index("c")
        s = lax.axis_index("s")
        wid = c * NS + s

        def zrow(r, carry):
            for k in range(LANES // 16):
                histo[r, pl.ds(k * 16, 16)] = jnp.zeros((16,), jnp.float32)
            return carry
        lax.fori_loop(0, hp_rows, zrow, 0)

        ones16 = jnp.ones((16,), jnp.float32)

        def step(i, carry):
            rb = wid * RW + i * IBD
            pltpu.sync_copy(dst_hbm.at[pl.ds(rb, IBD)], idxd)
            for r in range(IBD):
                for k in range(LANES // 16):
                    v = idxd[r, pl.ds(k * 16, 16)]
                    rowi = lax.shift_right_logical(v, 7)
                    coli = v & (LANES - 1)
                    plsc.addupdate_scatter(histo, [rowi, coli], ones16)
            return carry
        lax.fori_loop(0, RW // IBD, step, 0)

        pltpu.sync_copy(histo, out_hbm.at[wid])

    return deg


def _tc_first(N, Din, Dh, BR):
    def body(x_ref, w_ref, dp_ref, o_ref):
        deg = jnp.sum(dp_ref[...], axis=1, keepdims=True) + 1.0
        dis = lax.rsqrt(deg)
        h = jnp.dot(x_ref[...], w_ref[...], preferred_element_type=jnp.float32)
        o_ref[...] = h * dis

    return pl.pallas_call(
        body,
        grid=(N // BR,),
        in_specs=[
            pl.BlockSpec((BR, Din), lambda i: (i, 0)),
            pl.BlockSpec((Din, Dh), lambda i: (0, 0)),
            pl.BlockSpec((BR, NW), lambda i: (i, 0)),
        ],
        out_specs=pl.BlockSpec((BR, Dh), lambda i: (i, 0)),
        out_shape=jax.ShapeDtypeStruct((N, Dh), jnp.float32),
    )


def _tc_mid(N, Dh, Dn, BR, BPC):
    def body(p_ref, g_ref, dp_ref, w_ref, b_ref, o_ref):
        deg = jnp.sum(dp_ref[...], axis=1, keepdims=True) + 1.0
        dis = lax.rsqrt(deg)
        t = (p_ref[0] + g_ref[...]) * dis + b_ref[...]
        a = jnp.maximum(t, 0.0)
        h = jnp.dot(a, w_ref[...], preferred_element_type=jnp.float32)
        g2 = h * dis
        o_ref[...] = jnp.concatenate(
            [g2, jnp.zeros((g2.shape[0], Dh - Dn), jnp.float32)], axis=-1)

    return pl.pallas_call(
        body,
        grid=(N // BR,),
        in_specs=[
            pl.BlockSpec((1, BR, Dh), lambda i: (i // BPC, i % BPC, 0)),
            pl.BlockSpec((BR, Dh), lambda i: (i, 0)),
            pl.BlockSpec((BR, NW), lambda i: (i, 0)),
            pl.BlockSpec((Dh, Dn), lambda i: (0, 0)),
            pl.BlockSpec((1, Dh), lambda i: (0, 0)),
        ],
        out_specs=pl.BlockSpec((BR, Dh), lambda i: (i, 0)),
        out_shape=jax.ShapeDtypeStruct((N, Dh), jnp.float32),
    )


def _tc_last(N, Dp, Dh, Do, BR, BPC):
    def body(p_ref, g_ref, dp_ref, b_ref, wo_ref, bo_ref, o_ref):
        deg = jnp.sum(dp_ref[...], axis=1, keepdims=True) + 1.0
        dis = lax.rsqrt(deg)
        t = (p_ref[0][:, :Dh] + g_ref[...][:, :Dh]) * dis + b_ref[...]
        a = jnp.maximum(t, 0.0)
        o_ref[...] = (jnp.dot(a, wo_ref[...], preferred_element_type=jnp.float32)
                      + bo_ref[...])

    return pl.pallas_call(
        body,
        grid=(N // BR,),
        in_specs=[
            pl.BlockSpec((1, BR, Dp), lambda i: (i // BPC, i % BPC, 0)),
            pl.BlockSpec((BR, Dp), lambda i: (i, 0)),
            pl.BlockSpec((BR, NW), lambda i: (i, 0)),
            pl.BlockSpec((1, Dh), lambda i: (0, 0)),
            pl.BlockSpec((Dh, Do), lambda i: (0, 0)),
            pl.BlockSpec((1, Do), lambda i: (0, 0)),
        ],
        out_specs=pl.BlockSpec((BR, Do), lambda i: (i, 0)),
        out_shape=jax.ShapeDtypeStruct((N, Do), jnp.float32),
    )


def kernel(x, edge_index, W1, b1, W2, b2, Wo, bo):
    N, Din = x.shape
    E = edge_index.shape[1]
    Dh1 = W1.shape[1]
    Dh2 = W2.shape[1]
    Do = Wo.shape[1]

    r_raw = -(-E // LANES)
    gran = max(NS * IB, NW * IBD)
    R = -(-r_raw // gran) * gran
    e_pad = R * LANES - E
    src = edge_index[0].astype(jnp.int32)
    dst = edge_index[1].astype(jnp.int32)
    src_p = jnp.concatenate([src, jnp.zeros((e_pad,), jnp.int32)]).reshape(R, LANES)
    dst_p = jnp.concatenate([dst, jnp.full((e_pad,), N, jnp.int32)]).reshape(R, LANES)

    NH = -(-N // NC)
    TR0 = -(-NH // TRASH) * TRASH
    z_rows = 16
    acc_rows = -(-(TR0 + TRASH) // (NS * z_rows)) * NS * z_rows
    hp_rows = -(-(N + 1) // LANES)
    hp_rows += (-hp_rows) % 8

    BR = 1000 if (N // NC) % 1000 == 0 else 500
    BPC = (N // NC) // BR
    b1r = b1.reshape(1, Dh1)
    b2r = b2.reshape(1, Dh2)
    do_pad = max(8, Do)
    wo_p = jnp.zeros((Dh2, do_pad), jnp.float32).at[:, :Do].set(Wo)
    bo_p = jnp.zeros((1, do_pad), jnp.float32).at[0, :Do].set(bo)

    srcb_f, dstb_f, counts = _bin_sc(N, R, NH, TR0)(src_p, dst_p)
    srcb = srcb_f.reshape(NC, R, LANES)
    dstb = dstb_f.reshape(NC, R, LANES)
    deg_parts = _deg_sc(N, R, hp_rows)(dst_p)
    deg_t = jnp.transpose(deg_parts.reshape(NW, hp_rows * LANES))
    g1 = _tc_first(N, Din, Dh1, BR)(x, W1, deg_t)
    p1 = _spmm_sc(N, Dh1, R, NH, TR0, acc_rows, z_rows)(g1, srcb, dstb, counts)
    g2 = _tc_mid(N, Dh1, Dh2, BR, BPC)(p1, g1, deg_t, W2, b1r)
    p2 = _spmm_sc(N, Dh1, R, NH, TR0, acc_rows, z_rows)(g2, srcb, dstb, counts)
    out = _tc_last(N, Dh1, Dh2, do_pad, BR, BPC)(p2, g2, deg_t, b2r, wo_p, bo_p)
    return out[:, :Do]
